# Initial kernel scaffold; baseline (speedup 1.0000x reference)
import jax
import jax.numpy as jnp
from jax.experimental import pallas as pl


def kernel(nodes, edges, edge_index, edges_type, W_n_enc, b_n_enc, W_e_enc, b_e_enc, W_r1, W_root1, b_r1, W_gat, W_gat_e, att_src, att_dst, att_edge, b_gat, W_r2, W_root2, b_r2, W_dec, b_dec):
    raise NotImplementedError("write your pallas kernel here")


# speedup vs baseline: 1.3789x; 1.3789x over previous
import jax
import jax.numpy as jnp
from jax.experimental import pallas as pl

MAX_ACT = 5.0


def _decode_body(h_ref, w_ref, b_ref, o_ref):
    o_ref[...] = MAX_ACT * jnp.tanh(
        jnp.dot(h_ref[...], w_ref[...], preferred_element_type=jnp.float32)
        + b_ref[...]
    )


def kernel(nodes, edges, edge_index, edges_type, W_n_enc, b_n_enc, W_e_enc, b_e_enc, W_r1, W_root1, b_r1, W_gat, W_gat_e, att_src, att_dst, att_edge, b_gat, W_r2, W_root2, b_r2, W_dec, b_dec):
    N = nodes.shape[0]
    src = edge_index[0]
    dst = edge_index[1]
    et = edges_type

    n = jax.nn.relu(nodes @ W_n_enc + b_n_enc)
    e = jax.nn.relu(edges @ W_e_enc + b_e_enc)

    seg = dst * 2 + et
    src2t = src * 2 + et

    ones = jnp.ones((e.shape[0], 1), e.dtype)
    e_ext = jnp.concatenate([e, ones], axis=-1)
    Se = jax.ops.segment_sum(e_ext, seg, num_segments=2 * N)
    cnt = Se[:, 32].reshape(N, 2)
    inv = 1.0 / jnp.maximum(cnt, 1.0)
    enorm = Se[:, :32].reshape(N, 2, 32) * inv[:, :, None]

    def rgcn(h, W_rel, W_root, b):
        A = W_rel[:, :64, :]
        B = W_rel[:, 64:, :]
        hA = jnp.stack([h @ A[0], h @ A[1]], axis=1).reshape(2 * N, 64)
        hacc = jax.ops.segment_sum(hA[src2t], seg, num_segments=2 * N)
        hacc = hacc.reshape(N, 2, 64)
        agg = (hacc[:, 0] * inv[:, 0:1] + hacc[:, 1] * inv[:, 1:2]
               + enorm[:, 0] @ B[0] + enorm[:, 1] @ B[1])
        return h @ W_root + agg + b

    h = jax.nn.relu(rgcn(n, W_r1, W_root1, b_r1))

    hW = h @ W_gat
    s_src = hW @ att_src
    s_dst = hW @ att_dst
    ae = e @ (W_gat_e @ att_edge)
    a = s_src[src] + s_dst[dst] + ae
    a = jnp.where(a >= 0, a, 0.2 * a)
    ex = jnp.exp(a)
    numer = jax.ops.segment_sum(ex[:, None] * hW[src], dst, num_segments=N)
    denom = jax.ops.segment_sum(ex, dst, num_segments=N)
    h = jax.nn.relu(numer / (denom[:, None] + 1e-16) + b_gat)

    h = jax.nn.relu(rgcn(h, W_r2, W_root2, b_r2))

    out = pl.pallas_call(
        _decode_body,
        out_shape=jax.ShapeDtypeStruct((N, 1), jnp.float32),
    )(h, W_dec, b_dec.reshape(1, 1))
    return out


# speedup vs baseline: 18.1984x; 13.1974x over previous
import functools

import jax
import jax.numpy as jnp
from jax import lax
from jax.experimental import pallas as pl
from jax.experimental.pallas import tpu as pltpu
from jax.experimental.pallas import tpu_sc as plsc

F32 = jnp.float32
I32 = jnp.int32
MAX_ACT = 5.0
NC = 2
NS = 16
CH = 80


def _dot(a, b):
    return jnp.dot(a, b, preferred_element_type=F32)


def _tc_node1(nodes, W_n, b_n, A1):
    N = nodes.shape[0]
    Nb = 2000

    def body(nd_ref, wn_ref, bn_ref, a_ref, n_ref, ha_ref):
        nv = jnp.maximum(_dot(nd_ref[...], wn_ref[...]) + bn_ref[...], 0.0)
        n_ref[...] = nv
        ha_ref[0] = _dot(nv, a_ref[0])
        ha_ref[1] = _dot(nv, a_ref[1])

    return pl.pallas_call(
        body,
        grid=(N // Nb,),
        in_specs=[
            pl.BlockSpec((Nb, 4), lambda i: (i, 0)),
            pl.BlockSpec((4, 64), lambda i: (0, 0)),
            pl.BlockSpec((1, 64), lambda i: (0, 0)),
            pl.BlockSpec((2, 64, 64), lambda i: (0, 0, 0)),
        ],
        out_specs=[
            pl.BlockSpec((Nb, 64), lambda i: (i, 0)),
            pl.BlockSpec((2, Nb, 64), lambda i: (0, i, 0)),
        ],
        out_shape=[
            jax.ShapeDtypeStruct((N, 64), F32),
            jax.ShapeDtypeStruct((2, N, 64), F32),
        ],
    )(nodes, W_n, b_n.reshape(1, 64), A1)


def _tc_edge_feats(edges, We48, be48, Wge, att_e):
    E = edges.shape[0]
    Eb = 8000

    def body(ed_ref, we_ref, be_ref, wg_ref, ae_ref, eext_ref, aeo_ref):
        eext = jnp.maximum(_dot(ed_ref[...], we_ref[...]) + be_ref[...], 0.0)
        eext_ref[...] = eext
        vatt = _dot(wg_ref[...], ae_ref[...])
        aeo_ref[...] = _dot(eext[:, :32], vatt)

    return pl.pallas_call(
        body,
        grid=(E // Eb,),
        in_specs=[
            pl.BlockSpec((Eb, 2), lambda i: (i, 0)),
            pl.BlockSpec((2, 48), lambda i: (0, 0)),
            pl.BlockSpec((1, 48), lambda i: (0, 0)),
            pl.BlockSpec((32, 64), lambda i: (0, 0)),
            pl.BlockSpec((64, 1), lambda i: (0, 0)),
        ],
        out_specs=[
            pl.BlockSpec((Eb, 48), lambda i: (i, 0)),
            pl.BlockSpec((Eb, 1), lambda i: (i, 0)),
        ],
        out_shape=[
            jax.ShapeDtypeStruct((E, 48), F32),
            jax.ShapeDtypeStruct((E, 1), F32),
        ],
    )(edges, We48, be48, Wge, att_e.reshape(64, 1))


def _tc_indices(src2, dst2, et2, N):
    R = src2.shape[0]
    Rb = R // 5

    def body(s_ref, d_ref, t_ref, s2t_ref, seg_ref):
        toff = t_ref[...] * N
        s2t_ref[...] = s_ref[...] + toff
        seg_ref[...] = d_ref[...] + toff

    return pl.pallas_call(
        body,
        grid=(5,),
        in_specs=[pl.BlockSpec((Rb, 128), lambda i: (i, 0))] * 3,
        out_specs=[pl.BlockSpec((Rb, 128), lambda i: (i, 0))] * 2,
        out_shape=[
            jax.ShapeDtypeStruct((R, 128), I32),
            jax.ShapeDtypeStruct((R, 128), I32),
        ],
    )(src2, dst2, et2)


def _tc_inv(Separt, TWON):
    Mb = 2000

    def body(se_ref, inv_ref, en_ref):
        se = se_ref[0] + se_ref[1]
        iv = 1.0 / jnp.maximum(se[:, 32:33], 1.0)
        inv_ref[...] = iv
        en_ref[...] = se[:, 0:32] * iv

    return pl.pallas_call(
        body,
        grid=(TWON // Mb,),
        in_specs=[pl.BlockSpec((2, Mb, 48), lambda i: (0, i, 0))],
        out_specs=[
            pl.BlockSpec((Mb, 1), lambda i: (i, 0)),
            pl.BlockSpec((Mb, 32), lambda i: (i, 0)),
        ],
        out_shape=[
            jax.ShapeDtypeStruct((TWON, 1), F32),
            jax.ShapeDtypeStruct((TWON, 32), F32),
        ],
    )(Separt)


def _tc_node2(n, hacc1, enorm, B1, Wroot1, b_r1, Wgat, att_s, att_d):
    N = n.shape[0]
    Nb = 2000

    def body(n_ref, ha_ref, en_ref, b1_ref, wr_ref, br_ref, wg_ref,
             as_ref, ad_ref, h1_ref, g_ref, ss_ref, sd_ref):
        agg = (ha_ref[0] + ha_ref[1]
               + _dot(en_ref[0], b1_ref[0]) + _dot(en_ref[1], b1_ref[1]))
        h1 = jnp.maximum(_dot(n_ref[...], wr_ref[...]) + agg + br_ref[...], 0.0)
        h1_ref[...] = h1
        hw = _dot(h1, wg_ref[...])
        g_ref[:, 0:64] = hw
        g_ref[:, 64:80] = jnp.zeros_like(g_ref[:, 64:80])
        ss_ref[...] = _dot(hw, as_ref[...])
        sd_ref[...] = _dot(hw, ad_ref[...])

    return pl.pallas_call(
        body,
        grid=(N // Nb,),
        in_specs=[
            pl.BlockSpec((Nb, 64), lambda i: (i, 0)),
            pl.BlockSpec((2, Nb, 64), lambda i: (0, i, 0)),
            pl.BlockSpec((2, Nb, 32), lambda i: (0, i, 0)),
            pl.BlockSpec((2, 32, 64), lambda i: (0, 0, 0)),
            pl.BlockSpec((64, 64), lambda i: (0, 0)),
            pl.BlockSpec((1, 64), lambda i: (0, 0)),
            pl.BlockSpec((64, 64), lambda i: (0, 0)),
            pl.BlockSpec((64, 1), lambda i: (0, 0)),
            pl.BlockSpec((64, 1), lambda i: (0, 0)),
        ],
        out_specs=[
            pl.BlockSpec((Nb, 64), lambda i: (i, 0)),
            pl.BlockSpec((Nb, 80), lambda i: (i, 0)),
            pl.BlockSpec((Nb, 1), lambda i: (i, 0)),
            pl.BlockSpec((Nb, 1), lambda i: (i, 0)),
        ],
        out_shape=[
            jax.ShapeDtypeStruct((N, 64), F32),
            jax.ShapeDtypeStruct((N, 80), F32),
            jax.ShapeDtypeStruct((N, 1), F32),
            jax.ShapeDtypeStruct((N, 1), F32),
        ],
    )(n, hacc1, enorm, B1, Wroot1, b_r1.reshape(1, 64), Wgat,
      att_s.reshape(64, 1), att_d.reshape(64, 1))


def _tc_node3(gacc, h1, A2, b_gat):
    N = h1.shape[0]
    Nb = 2000

    def body(g_ref, h1_ref, a_ref, bg_ref, h2_ref, ha_ref):
        g = g_ref[0] + g_ref[1]
        h2 = jnp.maximum(g[:, 0:64] / (g[:, 64:65] + 1e-16) + bg_ref[...], 0.0)
        h2_ref[...] = h2
        ha_ref[0] = _dot(h2, a_ref[0])
        ha_ref[1] = _dot(h2, a_ref[1])

    return pl.pallas_call(
        body,
        grid=(N // Nb,),
        in_specs=[
            pl.BlockSpec((2, Nb, 80), lambda i: (0, i, 0)),
            pl.BlockSpec((Nb, 64), lambda i: (i, 0)),
            pl.BlockSpec((2, 64, 64), lambda i: (0, 0, 0)),
            pl.BlockSpec((1, 64), lambda i: (0, 0)),
        ],
        out_specs=[
            pl.BlockSpec((Nb, 64), lambda i: (i, 0)),
            pl.BlockSpec((2, Nb, 64), lambda i: (0, i, 0)),
        ],
        out_shape=[
            jax.ShapeDtypeStruct((N, 64), F32),
            jax.ShapeDtypeStruct((2, N, 64), F32),
        ],
    )(gacc, h1, A2, b_gat.reshape(1, 64))


def _tc_node4(h2, hacc2, enorm, B2, Wroot2, b_r2, W_dec, b_dec):
    N = h2.shape[0]
    Nb = 2000

    def body(h2_ref, ha_ref, en_ref, b2_ref, wr_ref, br_ref,
             wd_ref, bd_ref, o_ref):
        agg = (ha_ref[0] + ha_ref[1]
               + _dot(en_ref[0], b2_ref[0]) + _dot(en_ref[1], b2_ref[1]))
        h3 = jnp.maximum(_dot(h2_ref[...], wr_ref[...]) + agg + br_ref[...], 0.0)
        o_ref[...] = MAX_ACT * jnp.tanh(_dot(h3, wd_ref[...]) + bd_ref[...])

    return pl.pallas_call(
        body,
        grid=(N // Nb,),
        in_specs=[
            pl.BlockSpec((Nb, 64), lambda i: (i, 0)),
            pl.BlockSpec((2, Nb, 64), lambda i: (0, i, 0)),
            pl.BlockSpec((2, Nb, 32), lambda i: (0, i, 0)),
            pl.BlockSpec((2, 32, 64), lambda i: (0, 0, 0)),
            pl.BlockSpec((64, 64), lambda i: (0, 0)),
            pl.BlockSpec((1, 64), lambda i: (0, 0)),
            pl.BlockSpec((64, 1), lambda i: (0, 0)),
            pl.BlockSpec((1, 1), lambda i: (0, 0)),
        ],
        out_specs=[pl.BlockSpec((Nb, 1), lambda i: (i, 0))],
        out_shape=[jax.ShapeDtypeStruct((N, 1), F32)],
    )(h2, hacc2, enorm, B2, Wroot2, b_r2.reshape(1, 64),
      W_dec, b_dec.reshape(1, 1))[0]


def _mesh():
    return plsc.VectorSubcoreMesh(core_axis_name="c", subcore_axis_name="s",
                                  num_cores=NC, num_subcores=NS)


def _pad128(n):
    return (n + 127) // 128 * 128


def _sc_seg_scatter(e_ext, seg2d, zeros, nrows, width):
    E = e_ext.shape[0]
    EPW = E // (NC * NS)
    NCHK = EPW // CH
    RPT = nrows // NS

    @functools.partial(
        pl.kernel,
        out_type=jax.ShapeDtypeStruct((NC, nrows, width), F32),
        mesh=_mesh(),
        compiler_params=pltpu.CompilerParams(use_tc_tiling_on_sc=False, needs_layout_passes=False),
        scratch_types=[
            pltpu.VMEM((NCHK, CH), I32),
            pltpu.VMEM((CH, width), F32),
            pltpu.VMEM((CH, width), F32),
            pltpu.VMEM_SHARED((nrows, width), F32),
            pltpu.SemaphoreType.DMA,
            pltpu.SemaphoreType.DMA,
        ],
    )
    def kern(eext_hbm, seg_hbm, z_hbm, out_hbm, segv, rows0, rows1, acc,
             sem0, sem1):
        c = lax.axis_index("c")
        s = lax.axis_index("s")
        pltpu.sync_copy(z_hbm, acc.at[pl.ds(s * RPT, RPT)])
        pltpu.sync_copy(seg_hbm.at[c, s], segv)
        plsc.subcore_barrier()
        base = (c * NS + s) * EPW
        pltpu.async_copy(eext_hbm.at[pl.ds(base, CH)], rows0, sem0)

        def body(i, carry):
            c0 = 2 * i
            c1 = 2 * i + 1
            pltpu.async_copy(eext_hbm.at[pl.ds(base + c1 * CH, CH)], rows1,
                             sem1)
            pltpu.make_async_copy(eext_hbm.at[pl.ds(base + c0 * CH, CH)],
                                  rows0, sem0).wait()
            pltpu.sync_copy(rows0, acc.at[segv.at[c0]], add=True)

            @pl.when(c1 + 1 < NCHK)
            def _():
                pltpu.async_copy(eext_hbm.at[pl.ds(base + (c1 + 1) * CH, CH)],
                                 rows0, sem0)

            pltpu.make_async_copy(eext_hbm.at[pl.ds(base + c1 * CH, CH)],
                                  rows1, sem1).wait()
            pltpu.sync_copy(rows1, acc.at[segv.at[c1]], add=True)
            return carry

        lax.fori_loop(0, NCHK // 2, body, 0)
        plsc.subcore_barrier()
        pltpu.sync_copy(acc.at[pl.ds(s * RPT, RPT)],
                        out_hbm.at[c, pl.ds(s * RPT, RPT)])

    return kern(e_ext, seg2d, zeros)


def _sc_rgcn(tab, src2d, seg2d, dst2d, inv2n, zeros, nrows):
    TWON = tab.shape[0]
    E = src2d.size
    EPW = E // (NC * NS)
    NCHK = EPW // CH
    RPT = nrows // NS

    @functools.partial(
        pl.kernel,
        out_type=jax.ShapeDtypeStruct((NC, nrows, 64), F32),
        mesh=_mesh(),
        compiler_params=pltpu.CompilerParams(use_tc_tiling_on_sc=False, needs_layout_passes=False),
        scratch_types=[
            pltpu.VMEM((NCHK, CH), I32),
            pltpu.VMEM((NCHK, CH), I32),
            pltpu.VMEM((NCHK, CH), I32),
            pltpu.VMEM((TWON,), F32),
            pltpu.VMEM((CH,), F32),
            pltpu.VMEM((CH, 64), F32),
            pltpu.VMEM((CH, 64), F32),
            pltpu.VMEM_SHARED((nrows, 64), F32),
            pltpu.SemaphoreType.DMA,
            pltpu.SemaphoreType.DMA,
        ],
    )
    def kern(tab_hbm, src_hbm, seg_hbm, dst_hbm, inv_hbm, z_hbm, out_hbm,
             srcv, segv, dstv, invt, wb, rows0, rows1, acc, sem0, sem1):
        c = lax.axis_index("c")
        s = lax.axis_index("s")
        pltpu.sync_copy(z_hbm, acc.at[pl.ds(s * RPT, RPT)])
        pltpu.sync_copy(src_hbm.at[c, s], srcv)
        pltpu.sync_copy(seg_hbm.at[c, s], segv)
        pltpu.sync_copy(dst_hbm.at[c, s], dstv)
        pltpu.sync_copy(inv_hbm, invt)
        plsc.subcore_barrier()

        def compute(ci, rows):
            for j in range(CH // 16):
                segj = segv[ci, pl.ds(j * 16, 16)]
                wb[pl.ds(j * 16, 16)] = plsc.load_gather(invt, [segj])

            def scale(i, carry):
                w = plsc.load_gather(wb, [jnp.full((16,), i, I32)])
                for q in range(4):
                    rows[i, pl.ds(q * 16, 16)] = rows[i, pl.ds(q * 16, 16)] * w
                return carry

            lax.fori_loop(0, CH, scale, 0)

        pltpu.async_copy(tab_hbm.at[srcv.at[0]], rows0, sem0)

        def body(i, carry):
            c0 = 2 * i
            c1 = 2 * i + 1
            pltpu.async_copy(tab_hbm.at[srcv.at[c1]], rows1, sem1)
            pltpu.make_async_copy(tab_hbm.at[srcv.at[c0]], rows0, sem0).wait()
            compute(c0, rows0)
            pltpu.sync_copy(rows0, acc.at[dstv.at[c0]], add=True)

            @pl.when(c1 + 1 < NCHK)
            def _():
                pltpu.async_copy(tab_hbm.at[srcv.at[c1 + 1]], rows0, sem0)

            pltpu.make_async_copy(tab_hbm.at[srcv.at[c1]], rows1, sem1).wait()
            compute(c1, rows1)
            pltpu.sync_copy(rows1, acc.at[dstv.at[c1]], add=True)
            return carry

        lax.fori_loop(0, NCHK // 2, body, 0)
        plsc.subcore_barrier()
        pltpu.sync_copy(acc.at[pl.ds(s * RPT, RPT)],
                        out_hbm.at[c, pl.ds(s * RPT, RPT)])

    return kern(tab, src2d, seg2d, dst2d, inv2n, zeros)


def _sc_gat(Gtab, src2d, dst2d, ae2d, ssrc, sdst, zeros, nrows):
    N = Gtab.shape[0]
    E = src2d.size
    EPW = E // (NC * NS)
    NCHK = EPW // CH
    RPT = nrows // NS

    @functools.partial(
        pl.kernel,
        out_type=jax.ShapeDtypeStruct((NC, nrows, 80), F32),
        mesh=_mesh(),
        compiler_params=pltpu.CompilerParams(use_tc_tiling_on_sc=False, needs_layout_passes=False),
        scratch_types=[
            pltpu.VMEM((NCHK, CH), I32),
            pltpu.VMEM((NCHK, CH), I32),
            pltpu.VMEM((CH,), F32),
            pltpu.VMEM((CH,), F32),
            pltpu.VMEM((N,), F32),
            pltpu.VMEM((N,), F32),
            pltpu.VMEM((CH,), F32),
            pltpu.VMEM((CH, 80), F32),
            pltpu.VMEM((CH, 80), F32),
            pltpu.VMEM_SHARED((nrows, 80), F32),
            pltpu.SemaphoreType.DMA,
            pltpu.SemaphoreType.DMA,
            pltpu.SemaphoreType.DMA,
            pltpu.SemaphoreType.DMA,
        ],
    )
    def kern(g_hbm, src_hbm, dst_hbm, ae_hbm, ss_hbm, sd_hbm, z_hbm, out_hbm,
             srcv, dstv, ae0, ae1, sst, sdt, exb, rows0, rows1, acc,
             sem0, sem1, semA0, semA1):
        c = lax.axis_index("c")
        s = lax.axis_index("s")
        pltpu.sync_copy(z_hbm, acc.at[pl.ds(s * RPT, RPT)])
        pltpu.sync_copy(src_hbm.at[c, s], srcv)
        pltpu.sync_copy(dst_hbm.at[c, s], dstv)
        pltpu.sync_copy(ss_hbm, sst)
        pltpu.sync_copy(sd_hbm, sdt)
        plsc.subcore_barrier()
        onehot = jnp.where(lax.iota(I32, 16) == 0, 1.0, 0.0)

        def compute(ci, rows, aeb):
            for j in range(CH // 16):
                srcj = srcv[ci, pl.ds(j * 16, 16)]
                dstj = dstv[ci, pl.ds(j * 16, 16)]
                aej = aeb[pl.ds(j * 16, 16)]
                a = (plsc.load_gather(sst, [srcj])
                     + plsc.load_gather(sdt, [dstj]) + aej)
                a = jnp.where(a >= 0.0, a, 0.2 * a)
                exb[pl.ds(j * 16, 16)] = jnp.exp(a)

            def scale(i, carry):
                ex = plsc.load_gather(exb, [jnp.full((16,), i, I32)])
                for q in range(4):
                    rows[i, pl.ds(q * 16, 16)] = rows[i, pl.ds(q * 16, 16)] * ex
                rows[i, pl.ds(64, 16)] = onehot * ex
                return carry

            lax.fori_loop(0, CH, scale, 0)

        pltpu.async_copy(g_hbm.at[srcv.at[0]], rows0, sem0)
        pltpu.async_copy(ae_hbm.at[c, s, pl.ds(0, CH)], ae0, semA0)

        def body(i, carry):
            c0 = 2 * i
            c1 = 2 * i + 1
            pltpu.async_copy(g_hbm.at[srcv.at[c1]], rows1, sem1)
            pltpu.async_copy(ae_hbm.at[c, s, pl.ds(c1 * CH, CH)], ae1, semA1)
            pltpu.make_async_copy(g_hbm.at[srcv.at[c0]], rows0, sem0).wait()
            pltpu.make_async_copy(ae_hbm.at[c, s, pl.ds(c0 * CH, CH)], ae0, semA0).wait()
            compute(c0, rows0, ae0)
            pltpu.sync_copy(rows0, acc.at[dstv.at[c0]], add=True)

            @pl.when(c1 + 1 < NCHK)
            def _():
                pltpu.async_copy(g_hbm.at[srcv.at[c1 + 1]], rows0, sem0)
                pltpu.async_copy(ae_hbm.at[c, s, pl.ds((c1 + 1) * CH, CH)], ae0, semA0)

            pltpu.make_async_copy(g_hbm.at[srcv.at[c1]], rows1, sem1).wait()
            pltpu.make_async_copy(ae_hbm.at[c, s, pl.ds(c1 * CH, CH)], ae1, semA1).wait()
            compute(c1, rows1, ae1)
            pltpu.sync_copy(rows1, acc.at[dstv.at[c1]], add=True)
            return carry

        lax.fori_loop(0, NCHK // 2, body, 0)
        plsc.subcore_barrier()
        pltpu.sync_copy(acc.at[pl.ds(s * RPT, RPT)],
                        out_hbm.at[c, pl.ds(s * RPT, RPT)])

    return kern(Gtab, src2d, dst2d, ae2d, ssrc, sdst, zeros)


def kernel(nodes, edges, edge_index, edges_type, W_n_enc, b_n_enc, W_e_enc,
           b_e_enc, W_r1, W_root1, b_r1, W_gat, W_gat_e, att_src, att_dst,
           att_edge, b_gat, W_r2, W_root2, b_r2, W_dec, b_dec):
    N = nodes.shape[0]
    E = edges.shape[0]
    EPW = E // (NC * NS)
    NCHK = EPW // CH
    assert E == NC * NS * NCHK * CH and EPW % CH == 0
    assert N % NS == 0 and (2 * N) % NS == 0 and N % 2000 == 0

    A1, B1 = W_r1[:, :64, :], W_r1[:, 64:, :]
    A2, B2 = W_r2[:, :64, :], W_r2[:, 64:, :]
    We48 = jnp.concatenate([W_e_enc, jnp.zeros((2, 16), F32)], axis=1)
    be48 = jnp.concatenate(
        [b_e_enc, jnp.ones((1,), F32), jnp.zeros((15,), F32)]).reshape(1, 48)

    n, hA1 = _tc_node1(nodes, W_n_enc, b_n_enc, A1)
    e_ext, ae = _tc_edge_feats(edges, We48, be48, W_gat_e, att_edge)
    src2t, seg = _tc_indices(edge_index[0].reshape(E // 128, 128),
                             edge_index[1].reshape(E // 128, 128),
                             edges_type.reshape(E // 128, 128), N)

    shp = (NC, NS, NCHK, CH)
    seg2d = seg.reshape(shp)
    s2t2d = src2t.reshape(shp)
    src2d = edge_index[0].reshape(shp)
    dst2d = edge_index[1].reshape(shp)
    ae2d = ae.reshape(NC, NS, NCHK * CH)
    P2N = _pad128(2 * N)
    PN = _pad128(N)
    z48 = jnp.zeros((P2N // NS, 48), F32)
    z64 = jnp.zeros((PN // NS, 64), F32)
    z80 = jnp.zeros((PN // NS, 80), F32)

    Separt = _sc_seg_scatter(e_ext, seg2d, z48, P2N, 48)
    inv, enorm = _tc_inv(Separt[:, :2 * N], 2 * N)
    inv2n = inv.reshape(2 * N)
    enorm2 = enorm.reshape(2, N, 32)
    hacc1 = _sc_rgcn(hA1.reshape(2 * N, 64), s2t2d, seg2d, dst2d, inv2n,
                     z64, PN)
    h1, Gtab, ssrc, sdst = _tc_node2(
        n, hacc1[:, :N], enorm2,
        B1, W_root1, b_r1, W_gat, att_src, att_dst)
    gacc = _sc_gat(Gtab, src2d, dst2d, ae2d, ssrc.reshape(N),
                   sdst.reshape(N), z80, PN)
    h2, hA2 = _tc_node3(gacc[:, :N], h1, A2, b_gat)
    hacc2 = _sc_rgcn(hA2.reshape(2 * N, 64), s2t2d, seg2d, dst2d, inv2n,
                     z64, PN)
    out = _tc_node4(h2, hacc2[:, :N], enorm2,
                    B2, W_root2, b_r2, W_dec, b_dec)
    return out


# speedup vs baseline: 21.1798x; 1.1638x over previous
import functools

import jax
import jax.numpy as jnp
from jax import lax
from jax.experimental import pallas as pl
from jax.experimental.pallas import tpu as pltpu
from jax.experimental.pallas import tpu_sc as plsc

F32 = jnp.float32
I32 = jnp.int32
MAX_ACT = 5.0
NC = 2
NS = 16
CH = 80


def _dot(a, b):
    return jnp.dot(a, b, preferred_element_type=F32)


def _tc_node1(nodes, W_n, b_n, A1):
    N = nodes.shape[0]
    Nb = 2000

    def body(nd_ref, wn_ref, bn_ref, a_ref, n_ref, ha_ref):
        nv = jnp.maximum(_dot(nd_ref[...], wn_ref[...]) + bn_ref[...], 0.0)
        n_ref[...] = nv
        ha_ref[0] = _dot(nv, a_ref[0])
        ha_ref[1] = _dot(nv, a_ref[1])

    return pl.pallas_call(
        body,
        grid=(N // Nb,),
        in_specs=[
            pl.BlockSpec((Nb, 4), lambda i: (i, 0)),
            pl.BlockSpec((4, 64), lambda i: (0, 0)),
            pl.BlockSpec((1, 64), lambda i: (0, 0)),
            pl.BlockSpec((2, 64, 64), lambda i: (0, 0, 0)),
        ],
        out_specs=[
            pl.BlockSpec((Nb, 64), lambda i: (i, 0)),
            pl.BlockSpec((2, Nb, 64), lambda i: (0, i, 0)),
        ],
        out_shape=[
            jax.ShapeDtypeStruct((N, 64), F32),
            jax.ShapeDtypeStruct((2, N, 64), F32),
        ],
    )(nodes, W_n, b_n.reshape(1, 64), A1)


def _tc_edge_feats(edges, We48, be48, Wge, att_e):
    E = edges.shape[0]
    Eb = 8000

    def body(ed_ref, we_ref, be_ref, wg_ref, ae_ref, eext_ref, aeo_ref):
        eext = jnp.maximum(_dot(ed_ref[...], we_ref[...]) + be_ref[...], 0.0)
        eext_ref[...] = eext
        vatt = _dot(wg_ref[...], ae_ref[...])
        aeo_ref[...] = _dot(eext[:, :32], vatt)

    return pl.pallas_call(
        body,
        grid=(E // Eb,),
        in_specs=[
            pl.BlockSpec((Eb, 2), lambda i: (i, 0)),
            pl.BlockSpec((2, 48), lambda i: (0, 0)),
            pl.BlockSpec((1, 48), lambda i: (0, 0)),
            pl.BlockSpec((32, 64), lambda i: (0, 0)),
            pl.BlockSpec((64, 1), lambda i: (0, 0)),
        ],
        out_specs=[
            pl.BlockSpec((Eb, 48), lambda i: (i, 0)),
            pl.BlockSpec((Eb, 1), lambda i: (i, 0)),
        ],
        out_shape=[
            jax.ShapeDtypeStruct((E, 48), F32),
            jax.ShapeDtypeStruct((E, 1), F32),
        ],
    )(edges, We48, be48, Wge, att_e.reshape(64, 1))


def _tc_indices(src2, dst2, et2, N):
    R = src2.shape[0]
    Rb = R // 5

    def body(s_ref, d_ref, t_ref, s2t_ref, seg_ref):
        toff = t_ref[...] * N
        s2t_ref[...] = s_ref[...] + toff
        seg_ref[...] = d_ref[...] + toff

    return pl.pallas_call(
        body,
        grid=(5,),
        in_specs=[pl.BlockSpec((Rb, 128), lambda i: (i, 0))] * 3,
        out_specs=[pl.BlockSpec((Rb, 128), lambda i: (i, 0))] * 2,
        out_shape=[
            jax.ShapeDtypeStruct((R, 128), I32),
            jax.ShapeDtypeStruct((R, 128), I32),
        ],
    )(src2, dst2, et2)


def _tc_inv(Separt, TWON):
    Mb = 2000

    def body(se_ref, inv_ref, en_ref):
        se = se_ref[0] + se_ref[1]
        iv = 1.0 / jnp.maximum(se[:, 32:33], 1.0)
        inv_ref[...] = iv
        en_ref[...] = se[:, 0:32] * iv

    return pl.pallas_call(
        body,
        grid=(TWON // Mb,),
        in_specs=[pl.BlockSpec((2, Mb, 48), lambda i: (0, i, 0))],
        out_specs=[
            pl.BlockSpec((Mb, 1), lambda i: (i, 0)),
            pl.BlockSpec((Mb, 32), lambda i: (i, 0)),
        ],
        out_shape=[
            jax.ShapeDtypeStruct((TWON, 1), F32),
            jax.ShapeDtypeStruct((TWON, 32), F32),
        ],
    )(Separt)


def _tc_node2(n, hacc1, enorm, inv, B1, Wroot1, b_r1, Wgat, att_s, att_d):
    N = n.shape[0]
    Nb = 2000

    def body(n_ref, ha_ref, en_ref, iv_ref, b1_ref, wr_ref, br_ref, wg_ref,
             as_ref, ad_ref, h1_ref, g_ref, ss_ref, sd_ref):
        agg = ((ha_ref[0, 0] + ha_ref[1, 0]) * iv_ref[0]
               + (ha_ref[0, 1] + ha_ref[1, 1]) * iv_ref[1]
               + _dot(en_ref[0], b1_ref[0]) + _dot(en_ref[1], b1_ref[1]))
        h1 = jnp.maximum(_dot(n_ref[...], wr_ref[...]) + agg + br_ref[...], 0.0)
        h1_ref[...] = h1
        hw = _dot(h1, wg_ref[...])
        g_ref[:, 0:64] = hw
        g_ref[:, 64:80] = jnp.zeros_like(g_ref[:, 64:80])
        ss_ref[...] = _dot(hw, as_ref[...])
        sd_ref[...] = _dot(hw, ad_ref[...])

    return pl.pallas_call(
        body,
        grid=(N // Nb,),
        in_specs=[
            pl.BlockSpec((Nb, 64), lambda i: (i, 0)),
            pl.BlockSpec((2, 2, Nb, 64), lambda i: (0, 0, i, 0)),
            pl.BlockSpec((2, Nb, 32), lambda i: (0, i, 0)),
            pl.BlockSpec((2, Nb, 1), lambda i: (0, i, 0)),
            pl.BlockSpec((2, 32, 64), lambda i: (0, 0, 0)),
            pl.BlockSpec((64, 64), lambda i: (0, 0)),
            pl.BlockSpec((1, 64), lambda i: (0, 0)),
            pl.BlockSpec((64, 64), lambda i: (0, 0)),
            pl.BlockSpec((64, 1), lambda i: (0, 0)),
            pl.BlockSpec((64, 1), lambda i: (0, 0)),
        ],
        out_specs=[
            pl.BlockSpec((Nb, 64), lambda i: (i, 0)),
            pl.BlockSpec((Nb, 80), lambda i: (i, 0)),
            pl.BlockSpec((Nb, 1), lambda i: (i, 0)),
            pl.BlockSpec((Nb, 1), lambda i: (i, 0)),
        ],
        out_shape=[
            jax.ShapeDtypeStruct((N, 64), F32),
            jax.ShapeDtypeStruct((N, 80), F32),
            jax.ShapeDtypeStruct((N, 1), F32),
            jax.ShapeDtypeStruct((N, 1), F32),
        ],
    )(n, hacc1, enorm, inv, B1, Wroot1, b_r1.reshape(1, 64), Wgat,
      att_s.reshape(64, 1), att_d.reshape(64, 1))


def _tc_node3(gacc, h1, A2, b_gat):
    N = h1.shape[0]
    Nb = 2000

    def body(g_ref, h1_ref, a_ref, bg_ref, h2_ref, ha_ref):
        g = g_ref[0] + g_ref[1]
        h2 = jnp.maximum(g[:, 0:64] / (g[:, 64:65] + 1e-16) + bg_ref[...], 0.0)
        h2_ref[...] = h2
        ha_ref[0] = _dot(h2, a_ref[0])
        ha_ref[1] = _dot(h2, a_ref[1])

    return pl.pallas_call(
        body,
        grid=(N // Nb,),
        in_specs=[
            pl.BlockSpec((2, Nb, 80), lambda i: (0, i, 0)),
            pl.BlockSpec((Nb, 64), lambda i: (i, 0)),
            pl.BlockSpec((2, 64, 64), lambda i: (0, 0, 0)),
            pl.BlockSpec((1, 64), lambda i: (0, 0)),
        ],
        out_specs=[
            pl.BlockSpec((Nb, 64), lambda i: (i, 0)),
            pl.BlockSpec((2, Nb, 64), lambda i: (0, i, 0)),
        ],
        out_shape=[
            jax.ShapeDtypeStruct((N, 64), F32),
            jax.ShapeDtypeStruct((2, N, 64), F32),
        ],
    )(gacc, h1, A2, b_gat.reshape(1, 64))


def _tc_node4(h2, hacc2, enorm, inv, B2, Wroot2, b_r2, W_dec, b_dec):
    N = h2.shape[0]
    Nb = 2000

    def body(h2_ref, ha_ref, en_ref, iv_ref, b2_ref, wr_ref, br_ref,
             wd_ref, bd_ref, o_ref):
        agg = ((ha_ref[0, 0] + ha_ref[1, 0]) * iv_ref[0]
               + (ha_ref[0, 1] + ha_ref[1, 1]) * iv_ref[1]
               + _dot(en_ref[0], b2_ref[0]) + _dot(en_ref[1], b2_ref[1]))
        h3 = jnp.maximum(_dot(h2_ref[...], wr_ref[...]) + agg + br_ref[...], 0.0)
        o_ref[...] = MAX_ACT * jnp.tanh(_dot(h3, wd_ref[...]) + bd_ref[...])

    return pl.pallas_call(
        body,
        grid=(N // Nb,),
        in_specs=[
            pl.BlockSpec((Nb, 64), lambda i: (i, 0)),
            pl.BlockSpec((2, 2, Nb, 64), lambda i: (0, 0, i, 0)),
            pl.BlockSpec((2, Nb, 32), lambda i: (0, i, 0)),
            pl.BlockSpec((2, Nb, 1), lambda i: (0, i, 0)),
            pl.BlockSpec((2, 32, 64), lambda i: (0, 0, 0)),
            pl.BlockSpec((64, 64), lambda i: (0, 0)),
            pl.BlockSpec((1, 64), lambda i: (0, 0)),
            pl.BlockSpec((64, 1), lambda i: (0, 0)),
            pl.BlockSpec((1, 1), lambda i: (0, 0)),
        ],
        out_specs=[pl.BlockSpec((Nb, 1), lambda i: (i, 0))],
        out_shape=[jax.ShapeDtypeStruct((N, 1), F32)],
    )(h2, hacc2, enorm, inv, B2, Wroot2, b_r2.reshape(1, 64),
      W_dec, b_dec.reshape(1, 1))[0]


def _mesh():
    return plsc.VectorSubcoreMesh(core_axis_name="c", subcore_axis_name="s",
                                  num_cores=NC, num_subcores=NS)


def _pad128(n):
    return (n + 127) // 128 * 128


def _sc_seg_scatter(e_ext, seg2d, zeros, nrows, width):
    E = e_ext.shape[0]
    EPW = E // (NC * NS)
    NCHK = EPW // CH
    RPT = nrows // NS

    @functools.partial(
        pl.kernel,
        out_type=jax.ShapeDtypeStruct((NC, nrows, width), F32),
        mesh=_mesh(),
        compiler_params=pltpu.CompilerParams(use_tc_tiling_on_sc=False, needs_layout_passes=False),
        scratch_types=[
            pltpu.VMEM((NCHK, CH), I32),
            pltpu.VMEM((CH, width), F32),
            pltpu.VMEM((CH, width), F32),
            pltpu.VMEM_SHARED((nrows, width), F32),
            pltpu.SemaphoreType.DMA,
            pltpu.SemaphoreType.DMA,
        ],
    )
    def kern(eext_hbm, seg_hbm, z_hbm, out_hbm, segv, rows0, rows1, acc,
             sem0, sem1):
        c = lax.axis_index("c")
        s = lax.axis_index("s")
        pltpu.sync_copy(z_hbm, acc.at[pl.ds(s * RPT, RPT)])
        pltpu.sync_copy(seg_hbm.at[c, s], segv)
        plsc.subcore_barrier()
        base = (c * NS + s) * EPW
        pltpu.async_copy(eext_hbm.at[pl.ds(base, CH)], rows0, sem0)

        def body(i, carry):
            c0 = 2 * i
            c1 = 2 * i + 1
            pltpu.async_copy(eext_hbm.at[pl.ds(base + c1 * CH, CH)], rows1,
                             sem1)
            pltpu.make_async_copy(eext_hbm.at[pl.ds(base + c0 * CH, CH)],
                                  rows0, sem0).wait()
            pltpu.sync_copy(rows0, acc.at[segv.at[c0]], add=True)

            @pl.when(c1 + 1 < NCHK)
            def _():
                pltpu.async_copy(eext_hbm.at[pl.ds(base + (c1 + 1) * CH, CH)],
                                 rows0, sem0)

            pltpu.make_async_copy(eext_hbm.at[pl.ds(base + c1 * CH, CH)],
                                  rows1, sem1).wait()
            pltpu.sync_copy(rows1, acc.at[segv.at[c1]], add=True)
            return carry

        lax.fori_loop(0, NCHK // 2, body, 0)
        plsc.subcore_barrier()
        pltpu.sync_copy(acc.at[pl.ds(s * RPT, RPT)],
                        out_hbm.at[c, pl.ds(s * RPT, RPT)])

    return kern(e_ext, seg2d, zeros)


def _sc_rgcn(tab, src2d, seg2d, zeros, nrows):
    E = src2d.size
    EPW = E // (NC * NS)
    NCHK = EPW // CH
    RPT = nrows // NS

    @functools.partial(
        pl.kernel,
        out_type=jax.ShapeDtypeStruct((NC, nrows, 64), F32),
        mesh=_mesh(),
        compiler_params=pltpu.CompilerParams(use_tc_tiling_on_sc=False, needs_layout_passes=False),
        scratch_types=[
            pltpu.VMEM((NCHK, CH), I32),
            pltpu.VMEM((NCHK, CH), I32),
            pltpu.VMEM((CH, 64), F32),
            pltpu.VMEM((CH, 64), F32),
            pltpu.VMEM_SHARED((nrows, 64), F32),
            pltpu.SemaphoreType.DMA,
            pltpu.SemaphoreType.DMA,
        ],
    )
    def kern(tab_hbm, src_hbm, seg_hbm, z_hbm, out_hbm,
             srcv, segv, rows0, rows1, acc, sem0, sem1):
        c = lax.axis_index("c")
        s = lax.axis_index("s")
        pltpu.sync_copy(z_hbm, acc.at[pl.ds(s * RPT, RPT)])
        pltpu.sync_copy(src_hbm.at[c, s], srcv)
        pltpu.sync_copy(seg_hbm.at[c, s], segv)
        plsc.subcore_barrier()
        pltpu.async_copy(tab_hbm.at[srcv.at[0]], rows0, sem0)

        def body(i, carry):
            c0 = 2 * i
            c1 = 2 * i + 1
            pltpu.async_copy(tab_hbm.at[srcv.at[c1]], rows1, sem1)
            pltpu.make_async_copy(tab_hbm.at[srcv.at[c0]], rows0, sem0).wait()
            pltpu.sync_copy(rows0, acc.at[segv.at[c0]], add=True)

            @pl.when(c1 + 1 < NCHK)
            def _():
                pltpu.async_copy(tab_hbm.at[srcv.at[c1 + 1]], rows0, sem0)

            pltpu.make_async_copy(tab_hbm.at[srcv.at[c1]], rows1, sem1).wait()
            pltpu.sync_copy(rows1, acc.at[segv.at[c1]], add=True)
            return carry

        lax.fori_loop(0, NCHK // 2, body, 0)
        plsc.subcore_barrier()
        pltpu.sync_copy(acc.at[pl.ds(s * RPT, RPT)],
                        out_hbm.at[c, pl.ds(s * RPT, RPT)])

    return kern(tab, src2d, seg2d, zeros)


def _sc_gat(Gtab, src2d, dst2d, ae2d, ssrc, sdst, zeros, nrows):
    N = Gtab.shape[0]
    E = src2d.size
    EPW = E // (NC * NS)
    NCHK = EPW // CH
    RPT = nrows // NS

    @functools.partial(
        pl.kernel,
        out_type=jax.ShapeDtypeStruct((NC, nrows, 80), F32),
        mesh=_mesh(),
        compiler_params=pltpu.CompilerParams(use_tc_tiling_on_sc=False, needs_layout_passes=False),
        scratch_types=[
            pltpu.VMEM((NCHK, CH), I32),
            pltpu.VMEM((NCHK, CH), I32),
            pltpu.VMEM((CH,), F32),
            pltpu.VMEM((CH,), F32),
            pltpu.VMEM((N,), F32),
            pltpu.VMEM((N,), F32),
            pltpu.VMEM((CH,), F32),
            pltpu.VMEM((CH, 80), F32),
            pltpu.VMEM((CH, 80), F32),
            pltpu.VMEM_SHARED((nrows, 80), F32),
            pltpu.SemaphoreType.DMA,
            pltpu.SemaphoreType.DMA,
            pltpu.SemaphoreType.DMA,
            pltpu.SemaphoreType.DMA,
        ],
    )
    def kern(g_hbm, src_hbm, dst_hbm, ae_hbm, ss_hbm, sd_hbm, z_hbm, out_hbm,
             srcv, dstv, ae0, ae1, sst, sdt, exb, rows0, rows1, acc,
             sem0, sem1, semA0, semA1):
        c = lax.axis_index("c")
        s = lax.axis_index("s")
        pltpu.sync_copy(z_hbm, acc.at[pl.ds(s * RPT, RPT)])
        pltpu.sync_copy(src_hbm.at[c, s], srcv)
        pltpu.sync_copy(dst_hbm.at[c, s], dstv)
        pltpu.sync_copy(ss_hbm, sst)
        pltpu.sync_copy(sd_hbm, sdt)
        plsc.subcore_barrier()
        onehot = jnp.where(lax.iota(I32, 16) == 0, 1.0, 0.0)

        def compute(ci, rows, aeb):
            for j in range(CH // 16):
                srcj = srcv[ci, pl.ds(j * 16, 16)]
                dstj = dstv[ci, pl.ds(j * 16, 16)]
                aej = aeb[pl.ds(j * 16, 16)]
                a = (plsc.load_gather(sst, [srcj])
                     + plsc.load_gather(sdt, [dstj]) + aej)
                a = jnp.where(a >= 0.0, a, 0.2 * a)
                exb[pl.ds(j * 16, 16)] = jnp.exp(a)

            def scale(i, carry):
                ex = plsc.load_gather(exb, [jnp.full((16,), i, I32)])
                for q in range(4):
                    rows[i, pl.ds(q * 16, 16)] = rows[i, pl.ds(q * 16, 16)] * ex
                rows[i, pl.ds(64, 16)] = onehot * ex
                return carry

            lax.fori_loop(0, CH, scale, 0, unroll=8)

        pltpu.async_copy(g_hbm.at[srcv.at[0]], rows0, sem0)
        pltpu.async_copy(ae_hbm.at[c, s, pl.ds(0, CH)], ae0, semA0)

        def body(i, carry):
            c0 = 2 * i
            c1 = 2 * i + 1
            pltpu.async_copy(g_hbm.at[srcv.at[c1]], rows1, sem1)
            pltpu.async_copy(ae_hbm.at[c, s, pl.ds(c1 * CH, CH)], ae1, semA1)
            pltpu.make_async_copy(g_hbm.at[srcv.at[c0]], rows0, sem0).wait()
            pltpu.make_async_copy(ae_hbm.at[c, s, pl.ds(c0 * CH, CH)], ae0, semA0).wait()
            compute(c0, rows0, ae0)
            pltpu.sync_copy(rows0, acc.at[dstv.at[c0]], add=True)

            @pl.when(c1 + 1 < NCHK)
            def _():
                pltpu.async_copy(g_hbm.at[srcv.at[c1 + 1]], rows0, sem0)
                pltpu.async_copy(ae_hbm.at[c, s, pl.ds((c1 + 1) * CH, CH)], ae0, semA0)

            pltpu.make_async_copy(g_hbm.at[srcv.at[c1]], rows1, sem1).wait()
            pltpu.make_async_copy(ae_hbm.at[c, s, pl.ds(c1 * CH, CH)], ae1, semA1).wait()
            compute(c1, rows1, ae1)
            pltpu.sync_copy(rows1, acc.at[dstv.at[c1]], add=True)
            return carry

        lax.fori_loop(0, NCHK // 2, body, 0)
        plsc.subcore_barrier()
        pltpu.sync_copy(acc.at[pl.ds(s * RPT, RPT)],
                        out_hbm.at[c, pl.ds(s * RPT, RPT)])

    return kern(Gtab, src2d, dst2d, ae2d, ssrc, sdst, zeros)


def kernel(nodes, edges, edge_index, edges_type, W_n_enc, b_n_enc, W_e_enc,
           b_e_enc, W_r1, W_root1, b_r1, W_gat, W_gat_e, att_src, att_dst,
           att_edge, b_gat, W_r2, W_root2, b_r2, W_dec, b_dec):
    N = nodes.shape[0]
    E = edges.shape[0]
    EPW = E // (NC * NS)
    NCHK = EPW // CH
    assert E == NC * NS * NCHK * CH and EPW % CH == 0
    assert N % NS == 0 and (2 * N) % NS == 0 and N % 2000 == 0

    A1, B1 = W_r1[:, :64, :], W_r1[:, 64:, :]
    A2, B2 = W_r2[:, :64, :], W_r2[:, 64:, :]
    We48 = jnp.concatenate([W_e_enc, jnp.zeros((2, 16), F32)], axis=1)
    be48 = jnp.concatenate(
        [b_e_enc, jnp.ones((1,), F32), jnp.zeros((15,), F32)]).reshape(1, 48)

    n, hA1 = _tc_node1(nodes, W_n_enc, b_n_enc, A1)
    e_ext, ae = _tc_edge_feats(edges, We48, be48, W_gat_e, att_edge)
    src2t, seg = _tc_indices(edge_index[0].reshape(E // 128, 128),
                             edge_index[1].reshape(E // 128, 128),
                             edges_type.reshape(E // 128, 128), N)

    shp = (NC, NS, NCHK, CH)
    seg2d = seg.reshape(shp)
    s2t2d = src2t.reshape(shp)
    src2d = edge_index[0].reshape(shp)
    dst2d = edge_index[1].reshape(shp)
    ae2d = ae.reshape(NC, NS, NCHK * CH)
    P2N = _pad128(2 * N)
    PN = _pad128(N)
    z48 = jnp.zeros((P2N // NS, 48), F32)
    z64 = jnp.zeros((P2N // NS, 64), F32)
    z80 = jnp.zeros((PN // NS, 80), F32)

    Separt = _sc_seg_scatter(e_ext, seg2d, z48, P2N, 48)
    inv, enorm = _tc_inv(Separt[:, :2 * N], 2 * N)
    inv2 = inv.reshape(2, N, 1)
    enorm2 = enorm.reshape(2, N, 32)
    hacc1 = _sc_rgcn(hA1.reshape(2 * N, 64), s2t2d, seg2d, z64, P2N)
    h1, Gtab, ssrc, sdst = _tc_node2(
        n, hacc1[:, :2 * N].reshape(NC, 2, N, 64), enorm2, inv2,
        B1, W_root1, b_r1, W_gat, att_src, att_dst)
    gacc = _sc_gat(Gtab, src2d, dst2d, ae2d, ssrc.reshape(N),
                   sdst.reshape(N), z80, PN)
    h2, hA2 = _tc_node3(gacc[:, :N], h1, A2, b_gat)
    hacc2 = _sc_rgcn(hA2.reshape(2 * N, 64), s2t2d, seg2d, z64, P2N)
    out = _tc_node4(h2, hacc2[:, :2 * N].reshape(NC, 2, N, 64), enorm2, inv2,
                    B2, W_root2, b_r2, W_dec, b_dec)
    return out
